# baseline (speedup 1.0000x reference)
import functools

import jax
import jax.numpy as jnp
from jax import lax
from jax.experimental import pallas as pl
from jax.experimental.pallas import tpu as pltpu
from jax.experimental.pallas import tpu_sc as plsc

_NC = 2
_NS = 16
_NW = _NC * _NS
_L = 16
_CH = 80
_PREC = lax.Precision.HIGHEST
_F32 = jnp.float32


def _dense_proj(x, W, a_s, a_d):
    N, D = x.shape
    H = W.shape[1]
    BN = 400
    assert N % BN == 0

    def body(x_ref, w_ref, s_ref, d_ref, h_ref, os_ref, od_ref):
        h = jnp.dot(x_ref[...], w_ref[...], precision=_PREC,
                    preferred_element_type=_F32)
        h_ref[...] = h
        os_ref[...] = jnp.dot(h, s_ref[...], precision=_PREC)
        od_ref[...] = jnp.dot(h, d_ref[...], precision=_PREC)

    return pl.pallas_call(
        body,
        grid=(N // BN,),
        in_specs=[pl.BlockSpec((BN, D), lambda i: (i, 0)),
                  pl.BlockSpec((D, H), lambda i: (0, 0)),
                  pl.BlockSpec((H, 1), lambda i: (0, 0)),
                  pl.BlockSpec((H, 1), lambda i: (0, 0))],
        out_specs=[pl.BlockSpec((BN, H), lambda i: (i, 0)),
                   pl.BlockSpec((BN, 1), lambda i: (i, 0)),
                   pl.BlockSpec((BN, 1), lambda i: (i, 0))],
        out_shape=[jax.ShapeDtypeStruct((N, H), _F32),
                   jax.ShapeDtypeStruct((N, 1), _F32),
                   jax.ShapeDtypeStruct((N, 1), _F32)],
    )(x, W, a_s.reshape(H, 1), a_d.reshape(H, 1))


def _norm_proj(acc, den, b, W, a_s, a_d):
    _, N, H = acc.shape
    BN = 400
    assert N % BN == 0

    def body(a_ref, d0_ref, d1_ref, b_ref, w_ref, s_ref, d_ref,
             h_ref, os_ref, od_ref):
        a = a_ref[0] + a_ref[1]
        dn = d0_ref[...] + d1_ref[...]
        xb = jnp.maximum(a / (dn + 1e-16) + b_ref[...], 0.0)
        h = jnp.dot(xb, w_ref[...], precision=_PREC,
                    preferred_element_type=_F32)
        h_ref[...] = h
        os_ref[...] = jnp.dot(h, s_ref[...], precision=_PREC)
        od_ref[...] = jnp.dot(h, d_ref[...], precision=_PREC)

    return pl.pallas_call(
        body,
        grid=(N // BN,),
        in_specs=[pl.BlockSpec((_NC, BN, H), lambda i: (0, i, 0)),
                  pl.BlockSpec((BN, 1), lambda i: (i, 0)),
                  pl.BlockSpec((BN, 1), lambda i: (i, 0)),
                  pl.BlockSpec((1, H), lambda i: (0, 0)),
                  pl.BlockSpec((H, H), lambda i: (0, 0)),
                  pl.BlockSpec((H, 1), lambda i: (0, 0)),
                  pl.BlockSpec((H, 1), lambda i: (0, 0))],
        out_specs=[pl.BlockSpec((BN, H), lambda i: (i, 0)),
                   pl.BlockSpec((BN, 1), lambda i: (i, 0)),
                   pl.BlockSpec((BN, 1), lambda i: (i, 0))],
        out_shape=[jax.ShapeDtypeStruct((N, H), _F32),
                   jax.ShapeDtypeStruct((N, 1), _F32),
                   jax.ShapeDtypeStruct((N, 1), _F32)],
    )(acc, den[0].reshape(N, 1), den[1].reshape(N, 1), b.reshape(1, H),
      W, a_s.reshape(H, 1), a_d.reshape(H, 1))


def _norm_final(acc, den, b):
    _, N, H = acc.shape
    BN = 400
    assert N % BN == 0

    def body(a_ref, d0_ref, d1_ref, b_ref, h_ref, cs_ref):
        a = a_ref[0] + a_ref[1]
        dn = d0_ref[...] + d1_ref[...]
        y = jnp.maximum(a / (dn + 1e-16) + b_ref[...], 0.0)
        h_ref[...] = y
        cs = jnp.broadcast_to(jnp.sum(y, axis=0, keepdims=True), (8, H))

        @pl.when(pl.program_id(0) == 0)
        def _init():
            cs_ref[...] = cs

        @pl.when(pl.program_id(0) != 0)
        def _accum():
            cs_ref[...] = cs_ref[...] + cs

    return pl.pallas_call(
        body,
        grid=(N // BN,),
        in_specs=[pl.BlockSpec((_NC, BN, H), lambda i: (0, i, 0)),
                  pl.BlockSpec((BN, 1), lambda i: (i, 0)),
                  pl.BlockSpec((BN, 1), lambda i: (i, 0)),
                  pl.BlockSpec((1, H), lambda i: (0, 0))],
        out_specs=[pl.BlockSpec((BN, H), lambda i: (i, 0)),
                   pl.BlockSpec((8, H), lambda i: (0, 0))],
        out_shape=[jax.ShapeDtypeStruct((N, H), _F32),
                   jax.ShapeDtypeStruct((8, H), _F32)],
    )(acc, den[0].reshape(N, 1), den[1].reshape(N, 1), b.reshape(1, H))


def _heads(hs, ht, colsum, n_nodes, Wp1, bp1, Wp2, bp2, Wv1, bv1, Wv2, bv2):
    M, H = hs.shape
    BM = 400
    assert M % BM == 0
    inv_n = float(1.0 / n_nodes)

    def body(s_ref, t_ref, w1_ref, b1_ref, w2_ref, b2_ref,
             cs_ref, v1_ref, c1_ref, v2_ref, c2_ref,
             p_ref, val_ref, lbuf, stat):
        j = pl.program_id(0)
        i = pl.program_id(1)

        @pl.when(j == 0)
        def _logits():
            w1 = w1_ref[...]
            mf = (jnp.dot(s_ref[...], w1[:H, :], precision=_PREC,
                          preferred_element_type=_F32)
                  + jnp.dot(t_ref[...], w1[H:, :], precision=_PREC,
                            preferred_element_type=_F32)
                  + b1_ref[...])
            mf = jnp.maximum(mf, 0.0)
            l = jnp.dot(mf, w2_ref[...], precision=_PREC,
                        preferred_element_type=_F32) + b2_ref[...]
            lbuf[pl.ds(i * BM, BM), :] = l
            bm = jnp.max(l)

            @pl.when(i == 0)
            def _first():
                stat[0] = bm
                stat[1] = jnp.sum(jnp.exp(l - bm))

            @pl.when(i != 0)
            def _rest():
                m_old = stat[0]
                m_new = jnp.maximum(m_old, bm)
                stat[1] = (stat[1] * jnp.exp(m_old - m_new)
                           + jnp.sum(jnp.exp(l - m_new)))
                stat[0] = m_new

        @pl.when(j == 1)
        def _emit():
            m = stat[0]
            s = stat[1]
            p_ref[...] = jnp.exp(lbuf[pl.ds(i * BM, BM), :] - m) / s

            @pl.when(i == 0)
            def _value():
                g = cs_ref[0:1, :] * inv_n
                t = jnp.maximum(
                    jnp.dot(g, v1_ref[...], precision=_PREC,
                            preferred_element_type=_F32) + c1_ref[...], 0.0)
                val_ref[...] = jnp.tanh(
                    jnp.dot(t, v2_ref[...], precision=_PREC,
                            preferred_element_type=_F32) + c2_ref[...])

    return pl.pallas_call(
        body,
        grid=(2, M // BM),
        in_specs=[pl.BlockSpec((BM, H), lambda j, i: (i, 0)),
                  pl.BlockSpec((BM, H), lambda j, i: (i, 0)),
                  pl.BlockSpec((2 * H, H), lambda j, i: (0, 0)),
                  pl.BlockSpec((1, H), lambda j, i: (0, 0)),
                  pl.BlockSpec((H, 1), lambda j, i: (0, 0)),
                  pl.BlockSpec((1, 1), lambda j, i: (0, 0)),
                  pl.BlockSpec((8, H), lambda j, i: (0, 0)),
                  pl.BlockSpec((H, H), lambda j, i: (0, 0)),
                  pl.BlockSpec((1, H), lambda j, i: (0, 0)),
                  pl.BlockSpec((H, 1), lambda j, i: (0, 0)),
                  pl.BlockSpec((1, 1), lambda j, i: (0, 0))],
        out_specs=[pl.BlockSpec((BM, 1), lambda j, i: (i, 0)),
                   pl.BlockSpec((1, 1), lambda j, i: (0, 0))],
        out_shape=[jax.ShapeDtypeStruct((M, 1), _F32),
                   jax.ShapeDtypeStruct((1, 1), _F32)],
        scratch_shapes=[pltpu.VMEM((M, 1), _F32),
                        pltpu.SMEM((2,), _F32)],
    )(hs, ht, Wp1, bp1.reshape(1, H), Wp2, bp2.reshape(1, 1),
      colsum, Wv1, bv1.reshape(1, H), Wv2, bv2.reshape(1, 1))


def _sc_edge_aggregate(h, asrc, adst, sd_flat, dst_flat):
    N, H = h.shape
    R = dst_flat.shape[0] // _CH
    assert R % _NW == 0
    RPT = R // _NW
    DP = -(-N // 640) * 640
    SLAB = DP // _NS
    assert SLAB % _CH == 0
    NB = SLAB // _CH
    G = _CH // _L

    EPT = RPT * _CH
    mesh = plsc.VectorSubcoreMesh(core_axis_name="c", subcore_axis_name="s")

    @functools.partial(
        pl.kernel,
        out_type=(jax.ShapeDtypeStruct((_NC, DP, H), _F32),
                  jax.ShapeDtypeStruct((_NC * DP,), _F32)),
        mesh=mesh,
        compiler_params=pltpu.CompilerParams(needs_layout_passes=False),
        scratch_types=[
            pltpu.VMEM_SHARED((DP, H), _F32),
            pltpu.VMEM_SHARED((DP,), _F32),
            [pltpu.VMEM((_CH, H), _F32)] * 2,
            [pltpu.VMEM((_CH, H), _F32)] * 2,
            [pltpu.VMEM((_CH,), _F32)] * 2,
            [pltpu.VMEM((_CH,), _F32)] * 2,
            [pltpu.VMEM((_CH,), _F32)] * 2,
            [pltpu.VMEM((2 * _CH,), jnp.int32)] * 2,
            [pltpu.VMEM((_CH,), jnp.int32)] * 2,
            [pltpu.SemaphoreType.DMA] * 2,
            [pltpu.SemaphoreType.DMA] * 2,
            [pltpu.SemaphoreType.DMA] * 2,
            [pltpu.SemaphoreType.DMA] * 2,
            [pltpu.SemaphoreType.DMA] * 2,
            [pltpu.SemaphoreType.DMA] * 2,
            [pltpu.SemaphoreType.DMA] * 2,
        ],
    )
    def k(h_hbm, as_hbm, ad_hbm, sd_hbm, dstf_hbm, acc_out,
          den_out, acc_s, den_s, grows, srows, exv, ase, ade,
          sdidx, dsts, gsem, asem, aesem, ssem, dsem, igsem, issem):
        cid = lax.axis_index("c")
        sid = lax.axis_index("s")
        wid = sid * _NC + cid
        base = wid * RPT

        for i in range(_CH):
            for q in range(H // _L):
                grows[0][i, pl.ds(q * _L, _L)] = jnp.zeros((_L,), _F32)
        for g in range(G):
            exv[0][pl.ds(g * _L, _L)] = jnp.zeros((_L,), _F32)
        for b in range(NB):
            sl = pl.ds(sid * SLAB + b * _CH, _CH)
            pltpu.sync_copy(grows[0], acc_s.at[sl, :])
            pltpu.sync_copy(exv[0], den_s.at[sl])
        plsc.subcore_barrier()

        def load_gidx(j, b):
            r = base + j
            pltpu.async_copy(sd_hbm.at[pl.ds(r * 2 * _CH, 2 * _CH)], sdidx[b],
                             igsem[b])

        def wait_gidx(b):
            pltpu.make_async_copy(sd_hbm.at[pl.ds(0, 2 * _CH)], sdidx[b],
                                  igsem[b]).wait()

        def issue_gathers(b):
            si = sdidx[b].at[pl.ds(0, _CH)]
            di = sdidx[b].at[pl.ds(_CH, _CH)]
            pltpu.async_copy(h_hbm.at[si], grows[b], gsem[b])
            pltpu.async_copy(as_hbm.at[si], ase[b], asem[b])
            pltpu.async_copy(ad_hbm.at[di], ade[b], aesem[b])

        def wait_gathers(b):
            si = sdidx[b].at[pl.ds(0, _CH)]
            di = sdidx[b].at[pl.ds(_CH, _CH)]
            pltpu.make_async_copy(h_hbm.at[si], grows[b], gsem[b]).wait()
            pltpu.make_async_copy(as_hbm.at[si], ase[b], asem[b]).wait()
            pltpu.make_async_copy(ad_hbm.at[di], ade[b], aesem[b]).wait()

        def issue_scatter(b):
            pltpu.async_copy(srows[b], acc_s.at[dsts[b]], ssem[b], add=True)
            pltpu.async_copy(exv[b], den_s.at[dsts[b]], dsem[b], add=True)

        def wait_scatter(b):
            pltpu.make_async_copy(srows[b], acc_s.at[dsts[b]], ssem[b]).wait()
            pltpu.make_async_copy(exv[b], den_s.at[dsts[b]], dsem[b]).wait()

        def compute(b):
            for g in range(G):
                e = ase[b][pl.ds(g * _L, _L)] + ade[b][pl.ds(g * _L, _L)]
                e = jnp.where(e >= 0, e, e * jnp.float32(0.2))
                ex = jnp.exp(e)
                exv[b][pl.ds(g * _L, _L)] = ex
                for jj in range(_L):
                    i = g * _L + jj
                    s = ex[jj]
                    for q in range(H // _L):
                        srows[b][i, pl.ds(q * _L, _L)] = (
                            grows[b][i, pl.ds(q * _L, _L)] * s)

        def substep(j, b):
            @pl.when(j + 1 < RPT)
            def _prefetch():
                wait_gidx(1 - b)
                issue_gathers(1 - b)

            @pl.when(j >= 2)
            def _drain():
                wait_scatter(b)
            wait_gathers(b)
            @pl.when(j + 2 < RPT)
            def _refill():
                load_gidx(j + 2, b)
            r = base + j
            pltpu.async_copy(dstf_hbm.at[pl.ds(r * _CH, _CH)], dsts[b],
                             issem[b])
            compute(b)
            pltpu.make_async_copy(dstf_hbm.at[pl.ds(0, _CH)], dsts[b],
                                  issem[b]).wait()
            issue_scatter(b)

        load_gidx(0, 0)
        load_gidx(1, 1)
        wait_gidx(0)
        issue_gathers(0)

        def pair(t, carry):
            substep(2 * t, 0)
            substep(2 * t + 1, 1)
            return carry

        lax.fori_loop(0, RPT // 2, pair, 0)
        if RPT % 2:
            substep(RPT - 1, 0)
        wait_scatter((RPT - 2) % 2)
        wait_scatter((RPT - 1) % 2)

        plsc.subcore_barrier()
        for b in range(NB):
            off = sid * SLAB + b * _CH
            sl = pl.ds(off, _CH)
            pltpu.sync_copy(acc_s.at[sl, :], acc_out.at[cid, sl, :])
            pltpu.sync_copy(den_s.at[sl], exv[0])
            pltpu.sync_copy(exv[0], den_out.at[pl.ds(cid * DP + off, _CH)])

    acc, den = k(h, asrc, adst, sd_flat, dst_flat)
    return acc[:, :N, :], den.reshape(_NC, DP)[:, :N]


def _sc_pair_gather(hf, lm2):
    N, H = hf.shape
    R = lm2.shape[0]
    TPW = -(-R // _NW)

    mesh = plsc.VectorSubcoreMesh(core_axis_name="c", subcore_axis_name="s")

    @functools.partial(
        pl.kernel,
        out_type=jax.ShapeDtypeStruct((R * _CH, H), _F32),
        mesh=mesh,
        scratch_types=[
            pltpu.VMEM((_CH,), jnp.int32),
            pltpu.VMEM((_CH, H), _F32),
            pltpu.SemaphoreType.DMA,
        ],
    )
    def k(hf_hbm, lm_hbm, out_hbm, idxv, rows, sem):
        cid = lax.axis_index("c")
        sid = lax.axis_index("s")
        wid = sid * _NC + cid

        def step(t, carry):
            r = t * _NW + wid

            @pl.when(r < R)
            def _do():
                pltpu.sync_copy(lm_hbm.at[r], idxv)
                pltpu.async_copy(hf_hbm.at[idxv], rows, sem).wait()
                pltpu.sync_copy(rows, out_hbm.at[pl.ds(r * _CH, _CH), :])
            return carry

        lax.fori_loop(0, TPW, step, 0)

    return k(hf, lm2)


def kernel(x, edge_index, legal_moves, W1, a_src1, a_dst1, b1,
           W2, a_src2, a_dst2, b2, Wp1, bp1, Wp2, bp2, Wv1, bv1, Wv2, bv2):
    N, D = x.shape
    H = W1.shape[1]
    E = edge_index.shape[1]
    M = legal_moves.shape[1]
    assert E % (_NW * _CH) == 0 and (2 * M) % _CH == 0

    src_flat = edge_index[0]
    dst_flat = edge_index[1]
    sd_flat = jnp.concatenate(
        [src_flat.reshape(E // _CH, _CH), dst_flat.reshape(E // _CH, _CH)],
        axis=1).reshape(2 * E)
    lm2 = legal_moves.reshape((2 * M) // _CH, _CH)

    h1, as1, ad1 = _dense_proj(x, W1, a_src1, a_dst1)
    acc1, den1 = _sc_edge_aggregate(h1, as1.reshape(N), ad1.reshape(N),
                                    sd_flat, dst_flat)
    h2, as2, ad2 = _norm_proj(acc1, den1, b1, W2, a_src2, a_dst2)
    acc2, den2 = _sc_edge_aggregate(h2, as2.reshape(N), ad2.reshape(N),
                                    sd_flat, dst_flat)
    hf, colsum = _norm_final(acc2, den2, b2)

    pairs = _sc_pair_gather(hf, lm2)
    hs = pairs[:M]
    ht = pairs[M:]

    probs, value = _heads(hs, ht, colsum, N, Wp1, bp1, Wp2, bp2,
                          Wv1, bv1, Wv2, bv2)
    return value, probs.reshape(M)


# speedup vs baseline: 1.1296x; 1.1296x over previous
import functools

import jax
import jax.numpy as jnp
from jax import lax
from jax.experimental import pallas as pl
from jax.experimental.pallas import tpu as pltpu
from jax.experimental.pallas import tpu_sc as plsc

_NC = 2
_NS = 16
_NW = _NC * _NS
_L = 16
_CH = 80
_PREC = lax.Precision.HIGHEST
_F32 = jnp.float32


def _dense_proj(x, W, a_s, a_d):
    N, D = x.shape
    H = W.shape[1]
    BN = 400
    assert N % BN == 0

    def body(x_ref, w_ref, s_ref, d_ref, h_ref, os_ref, od_ref):
        h = jnp.dot(x_ref[...], w_ref[...], precision=_PREC,
                    preferred_element_type=_F32)
        h_ref[...] = h
        os_ref[...] = jnp.dot(h, s_ref[...], precision=_PREC)
        od_ref[...] = jnp.dot(h, d_ref[...], precision=_PREC)

    return pl.pallas_call(
        body,
        grid=(N // BN,),
        in_specs=[pl.BlockSpec((BN, D), lambda i: (i, 0)),
                  pl.BlockSpec((D, H), lambda i: (0, 0)),
                  pl.BlockSpec((H, 1), lambda i: (0, 0)),
                  pl.BlockSpec((H, 1), lambda i: (0, 0))],
        out_specs=[pl.BlockSpec((BN, H), lambda i: (i, 0)),
                   pl.BlockSpec((BN, 1), lambda i: (i, 0)),
                   pl.BlockSpec((BN, 1), lambda i: (i, 0))],
        out_shape=[jax.ShapeDtypeStruct((N, H), _F32),
                   jax.ShapeDtypeStruct((N, 1), _F32),
                   jax.ShapeDtypeStruct((N, 1), _F32)],
    )(x, W, a_s.reshape(H, 1), a_d.reshape(H, 1))


def _norm_proj(acc, den, b, W, a_s, a_d):
    N = den.shape[1]
    H = acc.shape[2]
    BN = 400
    assert N % BN == 0

    def body(a_ref, d0_ref, d1_ref, b_ref, w_ref, s_ref, d_ref,
             h_ref, os_ref, od_ref):
        a = a_ref[0] + a_ref[1]
        dn = d0_ref[...] + d1_ref[...]
        xb = jnp.maximum(a / (dn + 1e-16) + b_ref[...], 0.0)
        h = jnp.dot(xb, w_ref[...], precision=_PREC,
                    preferred_element_type=_F32)
        h_ref[...] = h
        os_ref[...] = jnp.dot(h, s_ref[...], precision=_PREC)
        od_ref[...] = jnp.dot(h, d_ref[...], precision=_PREC)

    return pl.pallas_call(
        body,
        grid=(N // BN,),
        in_specs=[pl.BlockSpec((_NC, BN, H), lambda i: (0, i, 0)),
                  pl.BlockSpec((BN, 1), lambda i: (i, 0)),
                  pl.BlockSpec((BN, 1), lambda i: (i, 0)),
                  pl.BlockSpec((1, H), lambda i: (0, 0)),
                  pl.BlockSpec((H, H), lambda i: (0, 0)),
                  pl.BlockSpec((H, 1), lambda i: (0, 0)),
                  pl.BlockSpec((H, 1), lambda i: (0, 0))],
        out_specs=[pl.BlockSpec((BN, H), lambda i: (i, 0)),
                   pl.BlockSpec((BN, 1), lambda i: (i, 0)),
                   pl.BlockSpec((BN, 1), lambda i: (i, 0))],
        out_shape=[jax.ShapeDtypeStruct((N, H), _F32),
                   jax.ShapeDtypeStruct((N, 1), _F32),
                   jax.ShapeDtypeStruct((N, 1), _F32)],
    )(acc, den[0].reshape(N, 1), den[1].reshape(N, 1), b.reshape(1, H),
      W, a_s.reshape(H, 1), a_d.reshape(H, 1))


def _norm_final(acc, den, b):
    N = den.shape[1]
    H = acc.shape[2]
    BN = 400
    assert N % BN == 0

    def body(a_ref, d0_ref, d1_ref, b_ref, h_ref, cs_ref):
        a = a_ref[0] + a_ref[1]
        dn = d0_ref[...] + d1_ref[...]
        y = jnp.maximum(a / (dn + 1e-16) + b_ref[...], 0.0)
        h_ref[...] = y
        cs = jnp.broadcast_to(jnp.sum(y, axis=0, keepdims=True), (8, H))

        @pl.when(pl.program_id(0) == 0)
        def _init():
            cs_ref[...] = cs

        @pl.when(pl.program_id(0) != 0)
        def _accum():
            cs_ref[...] = cs_ref[...] + cs

    return pl.pallas_call(
        body,
        grid=(N // BN,),
        in_specs=[pl.BlockSpec((_NC, BN, H), lambda i: (0, i, 0)),
                  pl.BlockSpec((BN, 1), lambda i: (i, 0)),
                  pl.BlockSpec((BN, 1), lambda i: (i, 0)),
                  pl.BlockSpec((1, H), lambda i: (0, 0))],
        out_specs=[pl.BlockSpec((BN, H), lambda i: (i, 0)),
                   pl.BlockSpec((8, H), lambda i: (0, 0))],
        out_shape=[jax.ShapeDtypeStruct((N, H), _F32),
                   jax.ShapeDtypeStruct((8, H), _F32)],
    )(acc, den[0].reshape(N, 1), den[1].reshape(N, 1), b.reshape(1, H))


def _policy_logits(pairs, M, Wp1, bp1, Wp2, bp2):
    H = pairs.shape[1]
    BM = 400
    assert M % BM == 0
    OFF = M // BM

    def body(s_ref, t_ref, w1_ref, b1_ref, w2_ref, b2_ref, o_ref):
        w1 = w1_ref[...]
        mf = (jnp.dot(s_ref[...], w1[:H, :], precision=_PREC,
                      preferred_element_type=_F32)
              + jnp.dot(t_ref[...], w1[H:, :], precision=_PREC,
                        preferred_element_type=_F32)
              + b1_ref[...])
        mf = jnp.maximum(mf, 0.0)
        o_ref[...] = jnp.dot(mf, w2_ref[...], precision=_PREC,
                             preferred_element_type=_F32) + b2_ref[...]

    return pl.pallas_call(
        body,
        grid=(M // BM,),
        in_specs=[pl.BlockSpec((BM, H), lambda i: (i, 0)),
                  pl.BlockSpec((BM, H), lambda i: (i + OFF, 0)),
                  pl.BlockSpec((2 * H, H), lambda i: (0, 0)),
                  pl.BlockSpec((1, H), lambda i: (0, 0)),
                  pl.BlockSpec((H, 1), lambda i: (0, 0)),
                  pl.BlockSpec((1, 1), lambda i: (0, 0))],
        out_specs=[pl.BlockSpec((BM, 1), lambda i: (i, 0))],
        out_shape=[jax.ShapeDtypeStruct((M, 1), _F32)],
    )(pairs, pairs, Wp1, bp1.reshape(1, H), Wp2, bp2.reshape(1, 1))[0]


def _softmax_value(logits_row, colsum, n_nodes, Wv1, bv1, Wv2, bv2):
    _, M = logits_row.shape
    H = Wv1.shape[0]
    inv_n = float(1.0 / n_nodes)

    def body(l_ref, cs_ref, w1_ref, b1_ref, w2_ref, b2_ref, p_ref, v_ref):
        l = l_ref[...]
        m = jnp.max(l)
        p = jnp.exp(l - m)
        p_ref[...] = p / jnp.sum(p)
        g = cs_ref[0:1, :] * inv_n
        t = jnp.maximum(jnp.dot(g, w1_ref[...], precision=_PREC,
                                preferred_element_type=_F32) + b1_ref[...], 0.0)
        v_ref[...] = jnp.tanh(jnp.dot(t, w2_ref[...], precision=_PREC,
                                      preferred_element_type=_F32) + b2_ref[...])

    return pl.pallas_call(
        body,
        in_specs=[pl.BlockSpec((1, M), lambda: (0, 0)),
                  pl.BlockSpec((8, H), lambda: (0, 0)),
                  pl.BlockSpec((H, H), lambda: (0, 0)),
                  pl.BlockSpec((1, H), lambda: (0, 0)),
                  pl.BlockSpec((H, 1), lambda: (0, 0)),
                  pl.BlockSpec((1, 1), lambda: (0, 0))],
        out_specs=[pl.BlockSpec((1, M), lambda: (0, 0)),
                   pl.BlockSpec((1, 1), lambda: (0, 0))],
        out_shape=[jax.ShapeDtypeStruct((1, M), _F32),
                   jax.ShapeDtypeStruct((1, 1), _F32)],
    )(logits_row, colsum, Wv1, bv1.reshape(1, H), Wv2, bv2.reshape(1, 1))


def _sc_edge_aggregate(h, asrc, adst, src_flat, dst_flat):
    N, H = h.shape
    R = dst_flat.shape[0] // _CH
    assert R % _NW == 0
    RPT = R // _NW
    DP = -(-N // 640) * 640
    SLAB = DP // _NS
    assert SLAB % _CH == 0
    NB = SLAB // _CH
    G = _CH // _L

    EPT = RPT * _CH
    mesh = plsc.VectorSubcoreMesh(core_axis_name="c", subcore_axis_name="s")

    @functools.partial(
        pl.kernel,
        out_type=(jax.ShapeDtypeStruct((_NC, DP, H), _F32),
                  jax.ShapeDtypeStruct((_NC * DP,), _F32)),
        mesh=mesh,
        compiler_params=pltpu.CompilerParams(needs_layout_passes=False),
        scratch_types=[
            pltpu.VMEM_SHARED((DP, H), _F32),
            pltpu.VMEM_SHARED((DP,), _F32),
            [pltpu.VMEM((_CH, H), _F32)] * 2,
            [pltpu.VMEM((_CH, H), _F32)] * 2,
            [pltpu.VMEM((_CH,), _F32)] * 2,
            [pltpu.VMEM((_CH,), _F32)] * 2,
            [pltpu.VMEM((_CH,), _F32)] * 2,
            [pltpu.VMEM((_CH,), jnp.int32)] * 2,
            [pltpu.VMEM((_CH,), jnp.int32)] * 2,
            [pltpu.VMEM((_CH,), jnp.int32)] * 2,
            [pltpu.SemaphoreType.DMA] * 2,
            [pltpu.SemaphoreType.DMA] * 2,
            [pltpu.SemaphoreType.DMA] * 2,
            [pltpu.SemaphoreType.DMA] * 2,
            [pltpu.SemaphoreType.DMA] * 2,
            [pltpu.SemaphoreType.DMA] * 2,
            [pltpu.SemaphoreType.DMA] * 2,
        ],
    )
    def k(h_hbm, as_hbm, ad_hbm, src_hbm, dstf_hbm, acc_out,
          den_out, acc_s, den_s, grows, srows, exv, ase, ade,
          srcv, dstg, dsts, gsem, asem, aesem, ssem, dsem, igsem, issem):
        cid = lax.axis_index("c")
        sid = lax.axis_index("s")
        wid = sid * _NC + cid
        base = wid * RPT

        for i in range(_CH):
            for q in range(H // _L):
                grows[0][i, pl.ds(q * _L, _L)] = jnp.zeros((_L,), _F32)
        for g in range(G):
            exv[0][pl.ds(g * _L, _L)] = jnp.zeros((_L,), _F32)
        for b in range(NB):
            sl = pl.ds(sid * SLAB + b * _CH, _CH)
            pltpu.sync_copy(grows[0], acc_s.at[sl, :])
            pltpu.sync_copy(exv[0], den_s.at[sl])
        plsc.subcore_barrier()

        def load_gidx(j, b):
            r = base + j
            pltpu.async_copy(src_hbm.at[pl.ds(r * _CH, _CH)], srcv[b],
                             igsem[b])
            pltpu.async_copy(dstf_hbm.at[pl.ds(r * _CH, _CH)], dstg[b],
                             igsem[b])

        def wait_gidx(b):
            pltpu.make_async_copy(src_hbm.at[pl.ds(0, _CH)], srcv[b],
                                  igsem[b]).wait()
            pltpu.make_async_copy(dstf_hbm.at[pl.ds(0, _CH)], dstg[b],
                                  igsem[b]).wait()

        def issue_gathers(b):
            pltpu.async_copy(h_hbm.at[srcv[b]], grows[b], gsem[b])
            pltpu.async_copy(as_hbm.at[srcv[b]], ase[b], asem[b])
            pltpu.async_copy(ad_hbm.at[dstg[b]], ade[b], aesem[b])

        def wait_gathers(b):
            pltpu.make_async_copy(h_hbm.at[srcv[b]], grows[b], gsem[b]).wait()
            pltpu.make_async_copy(as_hbm.at[srcv[b]], ase[b], asem[b]).wait()
            pltpu.make_async_copy(ad_hbm.at[dstg[b]], ade[b], aesem[b]).wait()

        def issue_scatter(b):
            pltpu.async_copy(srows[b], acc_s.at[dsts[b]], ssem[b], add=True)
            pltpu.async_copy(exv[b], den_s.at[dsts[b]], dsem[b], add=True)

        def wait_scatter(b):
            pltpu.make_async_copy(srows[b], acc_s.at[dsts[b]], ssem[b]).wait()
            pltpu.make_async_copy(exv[b], den_s.at[dsts[b]], dsem[b]).wait()

        def compute(b):
            for g in range(G):
                e = ase[b][pl.ds(g * _L, _L)] + ade[b][pl.ds(g * _L, _L)]
                e = jnp.where(e >= 0, e, e * jnp.float32(0.2))
                ex = jnp.exp(e)
                exv[b][pl.ds(g * _L, _L)] = ex
                for jj in range(_L):
                    i = g * _L + jj
                    s = ex[jj]
                    for q in range(H // _L):
                        srows[b][i, pl.ds(q * _L, _L)] = (
                            grows[b][i, pl.ds(q * _L, _L)] * s)

        def substep(j, b):
            @pl.when(j + 1 < RPT)
            def _prefetch():
                wait_gidx(1 - b)
                issue_gathers(1 - b)

            @pl.when(j >= 2)
            def _drain():
                wait_scatter(b)
            wait_gathers(b)
            @pl.when(j + 2 < RPT)
            def _refill():
                load_gidx(j + 2, b)
            r = base + j
            pltpu.async_copy(dstf_hbm.at[pl.ds(r * _CH, _CH)], dsts[b],
                             issem[b])
            compute(b)
            pltpu.make_async_copy(dstf_hbm.at[pl.ds(0, _CH)], dsts[b],
                                  issem[b]).wait()
            issue_scatter(b)

        load_gidx(0, 0)
        load_gidx(1, 1)
        wait_gidx(0)
        issue_gathers(0)

        def pair(t, carry):
            substep(2 * t, 0)
            substep(2 * t + 1, 1)
            return carry

        lax.fori_loop(0, RPT // 2, pair, 0)
        if RPT % 2:
            substep(RPT - 1, 0)
        wait_scatter((RPT - 2) % 2)
        wait_scatter((RPT - 1) % 2)

        plsc.subcore_barrier()
        for b in range(NB):
            off = sid * SLAB + b * _CH
            sl = pl.ds(off, _CH)
            pltpu.sync_copy(acc_s.at[sl, :], acc_out.at[cid, sl, :])
            pltpu.sync_copy(den_s.at[sl], exv[0])
            pltpu.sync_copy(exv[0], den_out.at[pl.ds(cid * DP + off, _CH)])

    acc, den = k(h, asrc, adst, src_flat, dst_flat)
    return acc, den.reshape(_NC, DP)[:, :N]


def _sc_pair_gather(hf, lm2):
    N, H = hf.shape
    R = lm2.shape[0]
    TPW = -(-R // _NW)

    mesh = plsc.VectorSubcoreMesh(core_axis_name="c", subcore_axis_name="s")

    @functools.partial(
        pl.kernel,
        out_type=jax.ShapeDtypeStruct((R * _CH, H), _F32),
        mesh=mesh,
        scratch_types=[
            pltpu.VMEM((_CH,), jnp.int32),
            pltpu.VMEM((_CH, H), _F32),
            pltpu.SemaphoreType.DMA,
        ],
    )
    def k(hf_hbm, lm_hbm, out_hbm, idxv, rows, sem):
        cid = lax.axis_index("c")
        sid = lax.axis_index("s")
        wid = sid * _NC + cid

        def step(t, carry):
            r = t * _NW + wid

            @pl.when(r < R)
            def _do():
                pltpu.sync_copy(lm_hbm.at[r], idxv)
                pltpu.async_copy(hf_hbm.at[idxv], rows, sem).wait()
                pltpu.sync_copy(rows, out_hbm.at[pl.ds(r * _CH, _CH), :])
            return carry

        lax.fori_loop(0, TPW, step, 0)

    return k(hf, lm2)


def kernel(x, edge_index, legal_moves, W1, a_src1, a_dst1, b1,
           W2, a_src2, a_dst2, b2, Wp1, bp1, Wp2, bp2, Wv1, bv1, Wv2, bv2):
    N, D = x.shape
    H = W1.shape[1]
    E = edge_index.shape[1]
    M = legal_moves.shape[1]
    assert E % (_NW * _CH) == 0 and (2 * M) % _CH == 0

    src_flat = edge_index[0]
    dst_flat = edge_index[1]
    lm2 = legal_moves.reshape((2 * M) // _CH, _CH)

    h1, as1, ad1 = _dense_proj(x, W1, a_src1, a_dst1)
    acc1, den1 = _sc_edge_aggregate(h1, as1.reshape(N), ad1.reshape(N),
                                    src_flat, dst_flat)
    h2, as2, ad2 = _norm_proj(acc1, den1, b1, W2, a_src2, a_dst2)
    acc2, den2 = _sc_edge_aggregate(h2, as2.reshape(N), ad2.reshape(N),
                                    src_flat, dst_flat)
    hf, colsum = _norm_final(acc2, den2, b2)

    pairs = _sc_pair_gather(hf, lm2)
    logits = _policy_logits(pairs, M, Wp1, bp1, Wp2, bp2)
    probs_row, value = _softmax_value(logits.reshape(1, M), colsum, N,
                                      Wv1, bv1, Wv2, bv2)
    return value, probs_row.reshape(M)


# speedup vs baseline: 1.1387x; 1.0080x over previous
import functools

import jax
import jax.numpy as jnp
from jax import lax
from jax.experimental import pallas as pl
from jax.experimental.pallas import tpu as pltpu
from jax.experimental.pallas import tpu_sc as plsc

_NC = 2
_NS = 16
_NW = _NC * _NS
_L = 16
_CH = 80
_PREC = lax.Precision.HIGHEST
_F32 = jnp.float32


def _dense_proj(x, W, a_s, a_d):
    N, D = x.shape
    H = W.shape[1]
    BN = 400
    assert N % BN == 0

    def body(x_ref, w_ref, s_ref, d_ref, h_ref, os_ref, od_ref):
        h = jnp.dot(x_ref[...], w_ref[...], precision=_PREC,
                    preferred_element_type=_F32)
        h_ref[...] = h
        os_ref[...] = jnp.dot(h, s_ref[...], precision=_PREC)
        od_ref[...] = jnp.dot(h, d_ref[...], precision=_PREC)

    return pl.pallas_call(
        body,
        grid=(N // BN,),
        in_specs=[pl.BlockSpec((BN, D), lambda i: (i, 0)),
                  pl.BlockSpec((D, H), lambda i: (0, 0)),
                  pl.BlockSpec((H, 1), lambda i: (0, 0)),
                  pl.BlockSpec((H, 1), lambda i: (0, 0))],
        out_specs=[pl.BlockSpec((BN, H), lambda i: (i, 0)),
                   pl.BlockSpec((BN, 1), lambda i: (i, 0)),
                   pl.BlockSpec((BN, 1), lambda i: (i, 0))],
        out_shape=[jax.ShapeDtypeStruct((N, H), _F32),
                   jax.ShapeDtypeStruct((N, 1), _F32),
                   jax.ShapeDtypeStruct((N, 1), _F32)],
    )(x, W, a_s.reshape(H, 1), a_d.reshape(H, 1))


def _norm_proj(acc, den, b, W, a_s, a_d):
    N = den.shape[1]
    H = acc.shape[2]
    BN = 400
    assert N % BN == 0

    def body(a_ref, d0_ref, d1_ref, b_ref, w_ref, s_ref, d_ref,
             h_ref, os_ref, od_ref):
        a = a_ref[0] + a_ref[1]
        dn = d0_ref[...] + d1_ref[...]
        xb = jnp.maximum(a / (dn + 1e-16) + b_ref[...], 0.0)
        h = jnp.dot(xb, w_ref[...], precision=_PREC,
                    preferred_element_type=_F32)
        h_ref[...] = h
        os_ref[...] = jnp.dot(h, s_ref[...], precision=_PREC)
        od_ref[...] = jnp.dot(h, d_ref[...], precision=_PREC)

    return pl.pallas_call(
        body,
        grid=(N // BN,),
        in_specs=[pl.BlockSpec((_NC, BN, H), lambda i: (0, i, 0)),
                  pl.BlockSpec((BN, 1), lambda i: (i, 0)),
                  pl.BlockSpec((BN, 1), lambda i: (i, 0)),
                  pl.BlockSpec((1, H), lambda i: (0, 0)),
                  pl.BlockSpec((H, H), lambda i: (0, 0)),
                  pl.BlockSpec((H, 1), lambda i: (0, 0)),
                  pl.BlockSpec((H, 1), lambda i: (0, 0))],
        out_specs=[pl.BlockSpec((BN, H), lambda i: (i, 0)),
                   pl.BlockSpec((BN, 1), lambda i: (i, 0)),
                   pl.BlockSpec((BN, 1), lambda i: (i, 0))],
        out_shape=[jax.ShapeDtypeStruct((N, H), _F32),
                   jax.ShapeDtypeStruct((N, 1), _F32),
                   jax.ShapeDtypeStruct((N, 1), _F32)],
    )(acc, den[0].reshape(N, 1), den[1].reshape(N, 1), b.reshape(1, H),
      W, a_s.reshape(H, 1), a_d.reshape(H, 1))


def _norm_final(acc, den, b):
    N = den.shape[1]
    H = acc.shape[2]
    BN = 400
    assert N % BN == 0

    def body(a_ref, d0_ref, d1_ref, b_ref, h_ref, cs_ref):
        a = a_ref[0] + a_ref[1]
        dn = d0_ref[...] + d1_ref[...]
        y = jnp.maximum(a / (dn + 1e-16) + b_ref[...], 0.0)
        h_ref[...] = y
        cs = jnp.broadcast_to(jnp.sum(y, axis=0, keepdims=True), (8, H))

        @pl.when(pl.program_id(0) == 0)
        def _init():
            cs_ref[...] = cs

        @pl.when(pl.program_id(0) != 0)
        def _accum():
            cs_ref[...] = cs_ref[...] + cs

    return pl.pallas_call(
        body,
        grid=(N // BN,),
        in_specs=[pl.BlockSpec((_NC, BN, H), lambda i: (0, i, 0)),
                  pl.BlockSpec((BN, 1), lambda i: (i, 0)),
                  pl.BlockSpec((BN, 1), lambda i: (i, 0)),
                  pl.BlockSpec((1, H), lambda i: (0, 0))],
        out_specs=[pl.BlockSpec((BN, H), lambda i: (i, 0)),
                   pl.BlockSpec((8, H), lambda i: (0, 0))],
        out_shape=[jax.ShapeDtypeStruct((N, H), _F32),
                   jax.ShapeDtypeStruct((8, H), _F32)],
    )(acc, den[0].reshape(N, 1), den[1].reshape(N, 1), b.reshape(1, H))


def _policy_logits(pairs, M, Wp1, bp1, Wp2, bp2):
    H = pairs.shape[1]
    BM = 400
    assert M % BM == 0
    OFF = M // BM

    def body(s_ref, t_ref, w1_ref, b1_ref, w2_ref, b2_ref, o_ref):
        w1 = w1_ref[...]
        mf = (jnp.dot(s_ref[...], w1[:H, :], precision=_PREC,
                      preferred_element_type=_F32)
              + jnp.dot(t_ref[...], w1[H:, :], precision=_PREC,
                        preferred_element_type=_F32)
              + b1_ref[...])
        mf = jnp.maximum(mf, 0.0)
        o_ref[...] = jnp.dot(mf, w2_ref[...], precision=_PREC,
                             preferred_element_type=_F32) + b2_ref[...]

    return pl.pallas_call(
        body,
        grid=(M // BM,),
        in_specs=[pl.BlockSpec((BM, H), lambda i: (i, 0)),
                  pl.BlockSpec((BM, H), lambda i: (i + OFF, 0)),
                  pl.BlockSpec((2 * H, H), lambda i: (0, 0)),
                  pl.BlockSpec((1, H), lambda i: (0, 0)),
                  pl.BlockSpec((H, 1), lambda i: (0, 0)),
                  pl.BlockSpec((1, 1), lambda i: (0, 0))],
        out_specs=[pl.BlockSpec((BM, 1), lambda i: (i, 0))],
        out_shape=[jax.ShapeDtypeStruct((M, 1), _F32)],
    )(pairs, pairs, Wp1, bp1.reshape(1, H), Wp2, bp2.reshape(1, 1))[0]


def _softmax_value(logits_row, colsum, n_nodes, Wv1, bv1, Wv2, bv2):
    _, M = logits_row.shape
    H = Wv1.shape[0]
    inv_n = float(1.0 / n_nodes)

    def body(l_ref, cs_ref, w1_ref, b1_ref, w2_ref, b2_ref, p_ref, v_ref):
        l = l_ref[...]
        m = jnp.max(l)
        p = jnp.exp(l - m)
        p_ref[...] = p / jnp.sum(p)
        g = cs_ref[0:1, :] * inv_n
        t = jnp.maximum(jnp.dot(g, w1_ref[...], precision=_PREC,
                                preferred_element_type=_F32) + b1_ref[...], 0.0)
        v_ref[...] = jnp.tanh(jnp.dot(t, w2_ref[...], precision=_PREC,
                                      preferred_element_type=_F32) + b2_ref[...])

    return pl.pallas_call(
        body,
        in_specs=[pl.BlockSpec((1, M), lambda: (0, 0)),
                  pl.BlockSpec((8, H), lambda: (0, 0)),
                  pl.BlockSpec((H, H), lambda: (0, 0)),
                  pl.BlockSpec((1, H), lambda: (0, 0)),
                  pl.BlockSpec((H, 1), lambda: (0, 0)),
                  pl.BlockSpec((1, 1), lambda: (0, 0))],
        out_specs=[pl.BlockSpec((1, M), lambda: (0, 0)),
                   pl.BlockSpec((1, 1), lambda: (0, 0))],
        out_shape=[jax.ShapeDtypeStruct((1, M), _F32),
                   jax.ShapeDtypeStruct((1, 1), _F32)],
    )(logits_row, colsum, Wv1, bv1.reshape(1, H), Wv2, bv2.reshape(1, 1))


def _sc_edge_aggregate(h, asrc, adst, src_flat, dst_flat):
    N, H = h.shape
    R = dst_flat.shape[0] // _CH
    assert R % _NW == 0
    RPT = R // _NW
    DP = -(-N // 640) * 640
    SLAB = DP // _NS
    assert SLAB % _CH == 0
    NB = SLAB // _CH
    G = _CH // _L

    EPT = RPT * _CH
    mesh = plsc.VectorSubcoreMesh(core_axis_name="c", subcore_axis_name="s")

    @functools.partial(
        pl.kernel,
        out_type=(jax.ShapeDtypeStruct((_NC, DP, H), _F32),
                  jax.ShapeDtypeStruct((_NC * DP,), _F32)),
        mesh=mesh,
        compiler_params=pltpu.CompilerParams(needs_layout_passes=False),
        scratch_types=[
            pltpu.VMEM_SHARED((DP, H), _F32),
            pltpu.VMEM_SHARED((DP,), _F32),
            [pltpu.VMEM((_CH, H), _F32)] * 2,
            [pltpu.VMEM((_CH, H), _F32)] * 2,
            [pltpu.VMEM((_CH,), _F32)] * 2,
            [pltpu.VMEM((_CH,), _F32)] * 2,
            [pltpu.VMEM((_CH,), _F32)] * 2,
            [pltpu.VMEM((_CH,), jnp.int32)] * 2,
            [pltpu.VMEM((_CH,), jnp.int32)] * 2,
            [pltpu.VMEM((_CH,), jnp.int32)] * 2,
            [pltpu.SemaphoreType.DMA] * 2,
            [pltpu.SemaphoreType.DMA] * 2,
            [pltpu.SemaphoreType.DMA] * 2,
            [pltpu.SemaphoreType.DMA] * 2,
            [pltpu.SemaphoreType.DMA] * 2,
            [pltpu.SemaphoreType.DMA] * 2,
            [pltpu.SemaphoreType.DMA] * 2,
        ],
    )
    def k(h_hbm, as_hbm, ad_hbm, src_hbm, dstf_hbm, acc_out,
          den_out, acc_s, den_s, grows, srows, exv, ase, ade,
          srcv, dstg, dsts, gsem, asem, aesem, ssem, dsem, igsem, issem):
        cid = lax.axis_index("c")
        sid = lax.axis_index("s")
        wid = sid * _NC + cid
        base = wid * RPT

        def load_gidx(j, b):
            r = base + j
            pltpu.async_copy(src_hbm.at[pl.ds(r * _CH, _CH)], srcv[b],
                             igsem[b])
            pltpu.async_copy(dstf_hbm.at[pl.ds(r * _CH, _CH)], dstg[b],
                             igsem[b])

        def wait_gidx(b):
            pltpu.make_async_copy(src_hbm.at[pl.ds(0, _CH)], srcv[b],
                                  igsem[b]).wait()
            pltpu.make_async_copy(dstf_hbm.at[pl.ds(0, _CH)], dstg[b],
                                  igsem[b]).wait()

        def issue_gathers(b):
            pltpu.async_copy(h_hbm.at[srcv[b]], grows[b], gsem[b])
            pltpu.async_copy(as_hbm.at[srcv[b]], ase[b], asem[b])
            pltpu.async_copy(ad_hbm.at[dstg[b]], ade[b], aesem[b])

        def wait_gathers(b):
            pltpu.make_async_copy(h_hbm.at[srcv[b]], grows[b], gsem[b]).wait()
            pltpu.make_async_copy(as_hbm.at[srcv[b]], ase[b], asem[b]).wait()
            pltpu.make_async_copy(ad_hbm.at[dstg[b]], ade[b], aesem[b]).wait()

        def issue_scatter(b):
            pltpu.async_copy(srows[b], acc_s.at[dsts[b]], ssem[b], add=True)
            pltpu.async_copy(exv[b], den_s.at[dsts[b]], dsem[b], add=True)

        def wait_scatter(b):
            pltpu.make_async_copy(srows[b], acc_s.at[dsts[b]], ssem[b]).wait()
            pltpu.make_async_copy(exv[b], den_s.at[dsts[b]], dsem[b]).wait()

        def compute(b):
            for g in range(G):
                e = ase[b][pl.ds(g * _L, _L)] + ade[b][pl.ds(g * _L, _L)]
                e = jnp.where(e >= 0, e, e * jnp.float32(0.2))
                ex = jnp.exp(e)
                exv[b][pl.ds(g * _L, _L)] = ex
                for jj in range(_L):
                    i = g * _L + jj
                    s = ex[jj]
                    for q in range(H // _L):
                        srows[b][i, pl.ds(q * _L, _L)] = (
                            grows[b][i, pl.ds(q * _L, _L)] * s)

        def substep(j, b):
            @pl.when(j + 1 < RPT)
            def _prefetch():
                wait_gidx(1 - b)
                issue_gathers(1 - b)

            @pl.when(j >= 2)
            def _drain():
                wait_scatter(b)
            wait_gathers(b)
            @pl.when(j + 2 < RPT)
            def _refill():
                load_gidx(j + 2, b)
            r = base + j
            pltpu.async_copy(dstf_hbm.at[pl.ds(r * _CH, _CH)], dsts[b],
                             issem[b])
            compute(b)
            pltpu.make_async_copy(dstf_hbm.at[pl.ds(0, _CH)], dsts[b],
                                  issem[b]).wait()
            issue_scatter(b)

        load_gidx(0, 0)
        load_gidx(1, 1)
        for i in range(_CH):
            for q in range(H // _L):
                srows[0][i, pl.ds(q * _L, _L)] = jnp.zeros((_L,), _F32)
        for g in range(G):
            exv[0][pl.ds(g * _L, _L)] = jnp.zeros((_L,), _F32)
        wait_gidx(0)
        issue_gathers(0)
        for b in range(NB):
            sl = pl.ds(sid * SLAB + b * _CH, _CH)
            pltpu.sync_copy(srows[0], acc_s.at[sl, :])
            pltpu.sync_copy(exv[0], den_s.at[sl])
        plsc.subcore_barrier()

        def pair(t, carry):
            substep(2 * t, 0)
            substep(2 * t + 1, 1)
            return carry

        lax.fori_loop(0, RPT // 2, pair, 0)
        if RPT % 2:
            substep(RPT - 1, 0)
        wait_scatter((RPT - 2) % 2)
        wait_scatter((RPT - 1) % 2)

        plsc.subcore_barrier()
        for b in range(NB):
            off = sid * SLAB + b * _CH
            sl = pl.ds(off, _CH)
            pltpu.sync_copy(acc_s.at[sl, :], acc_out.at[cid, sl, :])
            pltpu.sync_copy(den_s.at[sl], exv[0])
            pltpu.sync_copy(exv[0], den_out.at[pl.ds(cid * DP + off, _CH)])

    acc, den = k(h, asrc, adst, src_flat, dst_flat)
    return acc, den.reshape(_NC, DP)[:, :N]


def _sc_pair_gather(hf, lm2):
    N, H = hf.shape
    R = lm2.shape[0]
    TPW = -(-R // _NW)

    mesh = plsc.VectorSubcoreMesh(core_axis_name="c", subcore_axis_name="s")

    @functools.partial(
        pl.kernel,
        out_type=jax.ShapeDtypeStruct((R * _CH, H), _F32),
        mesh=mesh,
        scratch_types=[
            pltpu.VMEM((_CH,), jnp.int32),
            pltpu.VMEM((_CH, H), _F32),
            pltpu.SemaphoreType.DMA,
        ],
    )
    def k(hf_hbm, lm_hbm, out_hbm, idxv, rows, sem):
        cid = lax.axis_index("c")
        sid = lax.axis_index("s")
        wid = sid * _NC + cid

        def step(t, carry):
            r = t * _NW + wid

            @pl.when(r < R)
            def _do():
                pltpu.sync_copy(lm_hbm.at[r], idxv)
                pltpu.async_copy(hf_hbm.at[idxv], rows, sem).wait()
                pltpu.sync_copy(rows, out_hbm.at[pl.ds(r * _CH, _CH), :])
            return carry

        lax.fori_loop(0, TPW, step, 0)

    return k(hf, lm2)


def kernel(x, edge_index, legal_moves, W1, a_src1, a_dst1, b1,
           W2, a_src2, a_dst2, b2, Wp1, bp1, Wp2, bp2, Wv1, bv1, Wv2, bv2):
    N, D = x.shape
    H = W1.shape[1]
    E = edge_index.shape[1]
    M = legal_moves.shape[1]
    assert E % (_NW * _CH) == 0 and (2 * M) % _CH == 0

    src_flat = edge_index[0]
    dst_flat = edge_index[1]
    lm2 = legal_moves.reshape((2 * M) // _CH, _CH)

    h1, as1, ad1 = _dense_proj(x, W1, a_src1, a_dst1)
    acc1, den1 = _sc_edge_aggregate(h1, as1.reshape(N), ad1.reshape(N),
                                    src_flat, dst_flat)
    h2, as2, ad2 = _norm_proj(acc1, den1, b1, W2, a_src2, a_dst2)
    acc2, den2 = _sc_edge_aggregate(h2, as2.reshape(N), ad2.reshape(N),
                                    src_flat, dst_flat)
    hf, colsum = _norm_final(acc2, den2, b2)

    pairs = _sc_pair_gather(hf, lm2)
    logits = _policy_logits(pairs, M, Wp1, bp1, Wp2, bp2)
    probs_row, value = _softmax_value(logits.reshape(1, M), colsum, N,
                                      Wv1, bv1, Wv2, bv2)
    return value, probs_row.reshape(M)


# speedup vs baseline: 1.1880x; 1.0433x over previous
import functools

import jax
import jax.numpy as jnp
from jax import lax
from jax.experimental import pallas as pl
from jax.experimental.pallas import tpu as pltpu
from jax.experimental.pallas import tpu_sc as plsc

_NC = 2
_NS = 16
_NW = _NC * _NS
_L = 16
_CH = 80
_PREC = lax.Precision.DEFAULT
_F32 = jnp.float32


def _dense_proj(x, W, a_s, a_d):
    N, D = x.shape
    H = W.shape[1]
    BN = 400
    assert N % BN == 0

    def body(x_ref, w_ref, s_ref, d_ref, h_ref, os_ref, od_ref):
        h = jnp.dot(x_ref[...], w_ref[...], precision=_PREC,
                    preferred_element_type=_F32)
        h_ref[...] = h
        os_ref[...] = jnp.dot(h, s_ref[...], precision=_PREC)
        od_ref[...] = jnp.dot(h, d_ref[...], precision=_PREC)

    return pl.pallas_call(
        body,
        grid=(N // BN,),
        in_specs=[pl.BlockSpec((BN, D), lambda i: (i, 0)),
                  pl.BlockSpec((D, H), lambda i: (0, 0)),
                  pl.BlockSpec((H, 1), lambda i: (0, 0)),
                  pl.BlockSpec((H, 1), lambda i: (0, 0))],
        out_specs=[pl.BlockSpec((BN, H), lambda i: (i, 0)),
                   pl.BlockSpec((BN, 1), lambda i: (i, 0)),
                   pl.BlockSpec((BN, 1), lambda i: (i, 0))],
        out_shape=[jax.ShapeDtypeStruct((N, H), _F32),
                   jax.ShapeDtypeStruct((N, 1), _F32),
                   jax.ShapeDtypeStruct((N, 1), _F32)],
    )(x, W, a_s.reshape(H, 1), a_d.reshape(H, 1))


def _norm_proj(acc, den, b, W, a_s, a_d):
    N = den.shape[1]
    H = acc.shape[2]
    BN = 400
    assert N % BN == 0

    def body(a_ref, d0_ref, d1_ref, b_ref, w_ref, s_ref, d_ref,
             h_ref, os_ref, od_ref):
        a = a_ref[0] + a_ref[1]
        dn = d0_ref[...] + d1_ref[...]
        xb = jnp.maximum(a / (dn + 1e-16) + b_ref[...], 0.0)
        h = jnp.dot(xb, w_ref[...], precision=_PREC,
                    preferred_element_type=_F32)
        h_ref[...] = h
        os_ref[...] = jnp.dot(h, s_ref[...], precision=_PREC)
        od_ref[...] = jnp.dot(h, d_ref[...], precision=_PREC)

    return pl.pallas_call(
        body,
        grid=(N // BN,),
        in_specs=[pl.BlockSpec((_NC, BN, H), lambda i: (0, i, 0)),
                  pl.BlockSpec((BN, 1), lambda i: (i, 0)),
                  pl.BlockSpec((BN, 1), lambda i: (i, 0)),
                  pl.BlockSpec((1, H), lambda i: (0, 0)),
                  pl.BlockSpec((H, H), lambda i: (0, 0)),
                  pl.BlockSpec((H, 1), lambda i: (0, 0)),
                  pl.BlockSpec((H, 1), lambda i: (0, 0))],
        out_specs=[pl.BlockSpec((BN, H), lambda i: (i, 0)),
                   pl.BlockSpec((BN, 1), lambda i: (i, 0)),
                   pl.BlockSpec((BN, 1), lambda i: (i, 0))],
        out_shape=[jax.ShapeDtypeStruct((N, H), _F32),
                   jax.ShapeDtypeStruct((N, 1), _F32),
                   jax.ShapeDtypeStruct((N, 1), _F32)],
    )(acc, den[0].reshape(N, 1), den[1].reshape(N, 1), b.reshape(1, H),
      W, a_s.reshape(H, 1), a_d.reshape(H, 1))


def _norm_final(acc, den, b):
    N = den.shape[1]
    H = acc.shape[2]
    BN = 400
    assert N % BN == 0

    def body(a_ref, d0_ref, d1_ref, b_ref, h_ref, cs_ref):
        a = a_ref[0] + a_ref[1]
        dn = d0_ref[...] + d1_ref[...]
        y = jnp.maximum(a / (dn + 1e-16) + b_ref[...], 0.0)
        h_ref[...] = y
        cs = jnp.broadcast_to(jnp.sum(y, axis=0, keepdims=True), (8, H))

        @pl.when(pl.program_id(0) == 0)
        def _init():
            cs_ref[...] = cs

        @pl.when(pl.program_id(0) != 0)
        def _accum():
            cs_ref[...] = cs_ref[...] + cs

    return pl.pallas_call(
        body,
        grid=(N // BN,),
        in_specs=[pl.BlockSpec((_NC, BN, H), lambda i: (0, i, 0)),
                  pl.BlockSpec((BN, 1), lambda i: (i, 0)),
                  pl.BlockSpec((BN, 1), lambda i: (i, 0)),
                  pl.BlockSpec((1, H), lambda i: (0, 0))],
        out_specs=[pl.BlockSpec((BN, H), lambda i: (i, 0)),
                   pl.BlockSpec((8, H), lambda i: (0, 0))],
        out_shape=[jax.ShapeDtypeStruct((N, H), _F32),
                   jax.ShapeDtypeStruct((8, H), _F32)],
    )(acc, den[0].reshape(N, 1), den[1].reshape(N, 1), b.reshape(1, H))


def _policy_logits(pairs, M, Wp1, bp1, Wp2, bp2):
    H = pairs.shape[1]
    BM = 400
    assert M % BM == 0
    OFF = M // BM

    def body(s_ref, t_ref, w1_ref, b1_ref, w2_ref, b2_ref, o_ref):
        w1 = w1_ref[...]
        mf = (jnp.dot(s_ref[...], w1[:H, :], precision=_PREC,
                      preferred_element_type=_F32)
              + jnp.dot(t_ref[...], w1[H:, :], precision=_PREC,
                        preferred_element_type=_F32)
              + b1_ref[...])
        mf = jnp.maximum(mf, 0.0)
        o_ref[...] = jnp.dot(mf, w2_ref[...], precision=_PREC,
                             preferred_element_type=_F32) + b2_ref[...]

    return pl.pallas_call(
        body,
        grid=(M // BM,),
        in_specs=[pl.BlockSpec((BM, H), lambda i: (i, 0)),
                  pl.BlockSpec((BM, H), lambda i: (i + OFF, 0)),
                  pl.BlockSpec((2 * H, H), lambda i: (0, 0)),
                  pl.BlockSpec((1, H), lambda i: (0, 0)),
                  pl.BlockSpec((H, 1), lambda i: (0, 0)),
                  pl.BlockSpec((1, 1), lambda i: (0, 0))],
        out_specs=[pl.BlockSpec((BM, 1), lambda i: (i, 0))],
        out_shape=[jax.ShapeDtypeStruct((M, 1), _F32)],
    )(pairs, pairs, Wp1, bp1.reshape(1, H), Wp2, bp2.reshape(1, 1))[0]


def _softmax_value(logits_row, colsum, n_nodes, Wv1, bv1, Wv2, bv2):
    _, M = logits_row.shape
    H = Wv1.shape[0]
    inv_n = float(1.0 / n_nodes)

    def body(l_ref, cs_ref, w1_ref, b1_ref, w2_ref, b2_ref, p_ref, v_ref):
        l = l_ref[...]
        m = jnp.max(l)
        p = jnp.exp(l - m)
        p_ref[...] = p / jnp.sum(p)
        g = cs_ref[0:1, :] * inv_n
        t = jnp.maximum(jnp.dot(g, w1_ref[...], precision=_PREC,
                                preferred_element_type=_F32) + b1_ref[...], 0.0)
        v_ref[...] = jnp.tanh(jnp.dot(t, w2_ref[...], precision=_PREC,
                                      preferred_element_type=_F32) + b2_ref[...])

    return pl.pallas_call(
        body,
        in_specs=[pl.BlockSpec((1, M), lambda: (0, 0)),
                  pl.BlockSpec((8, H), lambda: (0, 0)),
                  pl.BlockSpec((H, H), lambda: (0, 0)),
                  pl.BlockSpec((1, H), lambda: (0, 0)),
                  pl.BlockSpec((H, 1), lambda: (0, 0)),
                  pl.BlockSpec((1, 1), lambda: (0, 0))],
        out_specs=[pl.BlockSpec((1, M), lambda: (0, 0)),
                   pl.BlockSpec((1, 1), lambda: (0, 0))],
        out_shape=[jax.ShapeDtypeStruct((1, M), _F32),
                   jax.ShapeDtypeStruct((1, 1), _F32)],
    )(logits_row, colsum, Wv1, bv1.reshape(1, H), Wv2, bv2.reshape(1, 1))


def _sc_edge_aggregate(h, asrc, adst, src_flat, dst_flat):
    N, H = h.shape
    R = dst_flat.shape[0] // _CH
    assert R % _NW == 0
    RPT = R // _NW
    DP = -(-N // 640) * 640
    SLAB = DP // _NS
    assert SLAB % _CH == 0
    NB = SLAB // _CH
    G = _CH // _L

    EPT = RPT * _CH
    mesh = plsc.VectorSubcoreMesh(core_axis_name="c", subcore_axis_name="s")

    @functools.partial(
        pl.kernel,
        out_type=(jax.ShapeDtypeStruct((_NC, DP, H), _F32),
                  jax.ShapeDtypeStruct((_NC * DP,), _F32)),
        mesh=mesh,
        compiler_params=pltpu.CompilerParams(needs_layout_passes=False,
                                             disable_bounds_checks=True),
        scratch_types=[
            pltpu.VMEM_SHARED((DP, H), _F32),
            pltpu.VMEM_SHARED((DP,), _F32),
            [pltpu.VMEM((_CH, H), _F32)] * 2,
            [pltpu.VMEM((_CH, H), _F32)] * 2,
            [pltpu.VMEM((_CH,), _F32)] * 2,
            [pltpu.VMEM((_CH,), _F32)] * 2,
            [pltpu.VMEM((_CH,), _F32)] * 2,
            [pltpu.VMEM((_CH,), jnp.int32)] * 2,
            [pltpu.VMEM((_CH,), jnp.int32)] * 2,
            [pltpu.VMEM((_CH,), jnp.int32)] * 2,
            [pltpu.SemaphoreType.DMA] * 2,
            [pltpu.SemaphoreType.DMA] * 2,
            [pltpu.SemaphoreType.DMA] * 2,
            [pltpu.SemaphoreType.DMA] * 2,
            [pltpu.SemaphoreType.DMA] * 2,
            [pltpu.SemaphoreType.DMA] * 2,
            [pltpu.SemaphoreType.DMA] * 2,
        ],
    )
    def k(h_hbm, as_hbm, ad_hbm, src_hbm, dstf_hbm, acc_out,
          den_out, acc_s, den_s, grows, srows, exv, ase, ade,
          srcv, dstg, dsts, gsem, asem, aesem, ssem, dsem, igsem, issem):
        cid = lax.axis_index("c")
        sid = lax.axis_index("s")
        wid = sid * _NC + cid
        base = wid * RPT

        def load_gidx(j, b):
            r = base + j
            pltpu.async_copy(src_hbm.at[pl.ds(r * _CH, _CH)], srcv[b],
                             igsem[b])
            pltpu.async_copy(dstf_hbm.at[pl.ds(r * _CH, _CH)], dstg[b],
                             igsem[b])

        def wait_gidx(b):
            pltpu.make_async_copy(src_hbm.at[pl.ds(0, _CH)], srcv[b],
                                  igsem[b]).wait()
            pltpu.make_async_copy(dstf_hbm.at[pl.ds(0, _CH)], dstg[b],
                                  igsem[b]).wait()

        def issue_gathers(b):
            pltpu.async_copy(h_hbm.at[srcv[b]], grows[b], gsem[b])
            pltpu.async_copy(as_hbm.at[srcv[b]], ase[b], asem[b])
            pltpu.async_copy(ad_hbm.at[dstg[b]], ade[b], aesem[b])

        def wait_gathers(b):
            pltpu.make_async_copy(h_hbm.at[srcv[b]], grows[b], gsem[b]).wait()
            pltpu.make_async_copy(as_hbm.at[srcv[b]], ase[b], asem[b]).wait()
            pltpu.make_async_copy(ad_hbm.at[dstg[b]], ade[b], aesem[b]).wait()

        def issue_scatter(b):
            pltpu.async_copy(srows[b], acc_s.at[dsts[b]], ssem[b], add=True)
            pltpu.async_copy(exv[b], den_s.at[dsts[b]], dsem[b], add=True)

        def wait_scatter(b):
            pltpu.make_async_copy(srows[b], acc_s.at[dsts[b]], ssem[b]).wait()
            pltpu.make_async_copy(exv[b], den_s.at[dsts[b]], dsem[b]).wait()

        def compute(b):
            for g in range(G):
                e = ase[b][pl.ds(g * _L, _L)] + ade[b][pl.ds(g * _L, _L)]
                e = jnp.where(e >= 0, e, e * jnp.float32(0.2))
                ex = jnp.exp(e)
                exv[b][pl.ds(g * _L, _L)] = ex
                for jj in range(_L):
                    i = g * _L + jj
                    s = ex[jj]
                    for q in range(H // _L):
                        srows[b][i, pl.ds(q * _L, _L)] = (
                            grows[b][i, pl.ds(q * _L, _L)] * s)

        def substep(j, b):
            @pl.when(j + 1 < RPT)
            def _prefetch():
                wait_gidx(1 - b)
                issue_gathers(1 - b)

            @pl.when(j >= 2)
            def _drain():
                wait_scatter(b)
            wait_gathers(b)
            @pl.when(j + 2 < RPT)
            def _refill():
                load_gidx(j + 2, b)
            r = base + j
            pltpu.async_copy(dstf_hbm.at[pl.ds(r * _CH, _CH)], dsts[b],
                             issem[b])
            compute(b)
            pltpu.make_async_copy(dstf_hbm.at[pl.ds(0, _CH)], dsts[b],
                                  issem[b]).wait()
            issue_scatter(b)

        load_gidx(0, 0)
        load_gidx(1, 1)
        for i in range(_CH):
            for q in range(H // _L):
                srows[0][i, pl.ds(q * _L, _L)] = jnp.zeros((_L,), _F32)
        for g in range(G):
            exv[0][pl.ds(g * _L, _L)] = jnp.zeros((_L,), _F32)
        wait_gidx(0)
        issue_gathers(0)
        for b in range(NB):
            sl = pl.ds(sid * SLAB + b * _CH, _CH)
            pltpu.sync_copy(srows[0], acc_s.at[sl, :])
            pltpu.sync_copy(exv[0], den_s.at[sl])
        plsc.subcore_barrier()

        def pair(t, carry):
            substep(2 * t, 0)
            substep(2 * t + 1, 1)
            return carry

        lax.fori_loop(0, RPT // 2, pair, 0)
        if RPT % 2:
            substep(RPT - 1, 0)
        wait_scatter((RPT - 2) % 2)
        wait_scatter((RPT - 1) % 2)

        plsc.subcore_barrier()
        for b in range(NB):
            off = sid * SLAB + b * _CH
            sl = pl.ds(off, _CH)
            pltpu.sync_copy(acc_s.at[sl, :], acc_out.at[cid, sl, :])
            pltpu.sync_copy(den_s.at[sl], exv[0])
            pltpu.sync_copy(exv[0], den_out.at[pl.ds(cid * DP + off, _CH)])

    acc, den = k(h, asrc, adst, src_flat, dst_flat)
    return acc, den.reshape(_NC, DP)[:, :N]


def _sc_pair_gather(hf, lm2):
    N, H = hf.shape
    R = lm2.shape[0]
    TPW = -(-R // _NW)

    mesh = plsc.VectorSubcoreMesh(core_axis_name="c", subcore_axis_name="s")

    @functools.partial(
        pl.kernel,
        out_type=jax.ShapeDtypeStruct((R * _CH, H), _F32),
        mesh=mesh,
        compiler_params=pltpu.CompilerParams(needs_layout_passes=False,
                                             disable_bounds_checks=True),
        scratch_types=[
            pltpu.VMEM((_CH,), jnp.int32),
            pltpu.VMEM((_CH, H), _F32),
            pltpu.SemaphoreType.DMA,
        ],
    )
    def k(hf_hbm, lm_hbm, out_hbm, idxv, rows, sem):
        cid = lax.axis_index("c")
        sid = lax.axis_index("s")
        wid = sid * _NC + cid

        def step(t, carry):
            r = t * _NW + wid

            @pl.when(r < R)
            def _do():
                pltpu.sync_copy(lm_hbm.at[r], idxv)
                pltpu.async_copy(hf_hbm.at[idxv], rows, sem).wait()
                pltpu.sync_copy(rows, out_hbm.at[pl.ds(r * _CH, _CH), :])
            return carry

        lax.fori_loop(0, TPW, step, 0)

    return k(hf, lm2)


def kernel(x, edge_index, legal_moves, W1, a_src1, a_dst1, b1,
           W2, a_src2, a_dst2, b2, Wp1, bp1, Wp2, bp2, Wv1, bv1, Wv2, bv2):
    N, D = x.shape
    H = W1.shape[1]
    E = edge_index.shape[1]
    M = legal_moves.shape[1]
    assert E % (_NW * _CH) == 0 and (2 * M) % _CH == 0

    src_flat = edge_index[0]
    dst_flat = edge_index[1]
    lm2 = legal_moves.reshape((2 * M) // _CH, _CH)

    h1, as1, ad1 = _dense_proj(x, W1, a_src1, a_dst1)
    acc1, den1 = _sc_edge_aggregate(h1, as1.reshape(N), ad1.reshape(N),
                                    src_flat, dst_flat)
    h2, as2, ad2 = _norm_proj(acc1, den1, b1, W2, a_src2, a_dst2)
    acc2, den2 = _sc_edge_aggregate(h2, as2.reshape(N), ad2.reshape(N),
                                    src_flat, dst_flat)
    hf, colsum = _norm_final(acc2, den2, b2)

    pairs = _sc_pair_gather(hf, lm2)
    logits = _policy_logits(pairs, M, Wp1, bp1, Wp2, bp2)
    probs_row, value = _softmax_value(logits.reshape(1, M), colsum, N,
                                      Wv1, bv1, Wv2, bv2)
    return value, probs_row.reshape(M)
